# baseline (device time: 17351 ns/iter reference)
import jax
import jax.numpy as jnp
from jax import lax
from jax.experimental import pallas as pl
from jax.experimental.pallas import tpu as pltpu

N_DEV = 32
N_BLK = 8
BIG = 1e9


def kernel(x):
    m_per, n = x.shape
    blk = m_per // N_BLK

    def body(x_ref, out_ref, local_ref, xbuf, recv_buf, cp_sems,
             send_sems, recv_sems):
        my_pos = lax.axis_index("i")

        def cp(b, slot):
            return pltpu.make_async_copy(
                x_ref.at[pl.ds(b * blk, blk), :], xbuf.at[slot],
                cp_sems.at[slot],
            )

        cp(0, 0).start()

        barrier_sem = pltpu.get_barrier_semaphore()
        for j in range(N_DEV - 1):
            t = (my_pos + 1 + j) % N_DEV
            pl.semaphore_signal(
                barrier_sem, inc=1,
                device_id=(t,), device_id_type=pl.DeviceIdType.MESH,
            )

        rows = lax.broadcasted_iota(jnp.int32, (blk, n), 0)
        vmax_l = None
        gidx_l = None
        for b in range(N_BLK):
            if b + 1 < N_BLK:
                cp(b + 1, (b + 1) % 2).start()
            cp(b, b % 2).wait()
            xv = xbuf[b % 2]
            bmax = jnp.max(xv, axis=0)
            bidx = jnp.min(
                jnp.where(xv == bmax[None, :], rows, jnp.int32(blk)), axis=0
            )
            gidx = (my_pos * m_per + b * blk + bidx).astype(jnp.float32)
            if b == 0:
                vmax_l, gidx_l = bmax, gidx
            else:
                take = bmax > vmax_l
                vmax_l = jnp.where(take, bmax, vmax_l)
                gidx_l = jnp.where(take, gidx, gidx_l)
        local_ref[0, :] = vmax_l
        local_ref[1, :] = gidx_l

        pl.semaphore_wait(barrier_sem, N_DEV - 1)

        sends = []
        for j in range(N_DEV - 1):
            t = (my_pos + 1 + j) % N_DEV
            rdma = pltpu.make_async_remote_copy(
                src_ref=local_ref,
                dst_ref=recv_buf.at[my_pos],
                send_sem=send_sems.at[j],
                recv_sem=recv_sems.at[my_pos],
                device_id=(t,),
                device_id_type=pl.DeviceIdType.MESH,
            )
            rdma.start()
            sends.append(rdma)

        for j in range(N_DEV - 1):
            s = (my_pos + 1 + j) % N_DEV
            recv = pltpu.make_async_remote_copy(
                src_ref=local_ref,
                dst_ref=recv_buf.at[s],
                send_sem=send_sems.at[j],
                recv_sem=recv_sems.at[s],
                device_id=(s,),
                device_id_type=pl.DeviceIdType.MESH,
            )
            recv.wait_recv()
        for rdma in sends:
            rdma.wait_send()

        v = recv_buf[:, 0, :]
        i = recv_buf[:, 1, :]
        slot = lax.broadcasted_iota(jnp.int32, (N_DEV, n), 0)
        mine = slot == my_pos
        v = jnp.where(mine, jnp.float32(-jnp.inf), v)
        i = jnp.where(mine, BIG, i)
        vmax = jnp.maximum(jnp.max(v, axis=0), vmax_l)
        cand_r = jnp.min(jnp.where(v == vmax[None, :], i, BIG), axis=0)
        cand_l = jnp.where(vmax_l == vmax, gidx_l, BIG)
        out_ref[0, :] = vmax
        out_ref[1, :] = jnp.minimum(cand_r, cand_l)

    return pl.pallas_call(
        body,
        out_shape=jax.ShapeDtypeStruct((2, n), jnp.float32),
        in_specs=[pl.BlockSpec(memory_space=pltpu.MemorySpace.HBM)],
        out_specs=pl.BlockSpec(memory_space=pltpu.VMEM),
        scratch_shapes=[
            pltpu.VMEM((2, n), jnp.float32),
            pltpu.VMEM((2, blk, n), jnp.float32),
            pltpu.VMEM((N_DEV, 2, n), jnp.float32),
            pltpu.SemaphoreType.DMA((2,)),
            pltpu.SemaphoreType.DMA((N_DEV - 1,)),
            pltpu.SemaphoreType.DMA((N_DEV,)),
        ],
        compiler_params=pltpu.CompilerParams(collective_id=0),
    )(x)
